# baseline (speedup 1.0000x reference)
import jax
import jax.numpy as jnp
from jax import lax
from jax.experimental import pallas as pl
from jax.experimental.pallas import tpu as pltpu
from jax.experimental.pallas import tpu_sc as plsc

VOCAB = 1000000
DIM = 64
BATCH = 16384
NNEG = 20

LOG2C = 14
C = 1 << LOG2C
CB = 2 * C
NBLK = (VOCAB + CB - 1) // CB
VR = NBLK * C
TROWS = 2 * VR

NC = 2
NS = 16
L = 16
NW = NC * NS
BPW = BATCH // NW
CHUNK = 32
NCHUNK = BPW // CHUNK
NIDX = CHUNK * NNEG


def _tr_kernel(x_ref, i_ref, o_ref):
    x = x_ref[...]
    ident = i_ref[...]
    dn = (((0,), (0,)), ((), ()))
    xcat = jnp.concatenate([x[:, 0:C], x[:, C:CB]], axis=0)
    o_ref[...] = lax.dot_general(xcat, ident, dn,
                                 preferred_element_type=jnp.float32)


def _relayout_table(w):
    ident = jnp.eye(2 * DIM, dtype=jnp.float32)
    w128 = pl.pallas_call(
        _tr_kernel,
        grid=(NBLK,),
        in_specs=[pl.BlockSpec((DIM, CB), lambda i: (0, i)),
                  pl.BlockSpec((2 * DIM, 128), lambda i: (0, 0))],
        out_specs=pl.BlockSpec((C, 128), lambda i: (i, 0)),
        out_shape=jax.ShapeDtypeStruct((VR, 128), jnp.float32),
        compiler_params=pltpu.CompilerParams(
            vmem_limit_bytes=100 * 1024 * 1024),
    )(w.T, ident)
    return w128.reshape(TROWS, DIM)


def _transform_ref(ref, n):
    def body(j, _):
        v = ref[pl.ds(j * L, L)]
        g = (((v >> (LOG2C + 1)) << (LOG2C + 1))
             | ((v & (C - 1)) << 1) | ((v >> LOG2C) & 1))
        ref[pl.ds(j * L, L)] = g
        return 0
    lax.fori_loop(0, n // L, body, 0)


def _sc1_body(ctx_hbm, neg_hbm, wc_hbm,
              embc_hbm, negacc_hbm,
              ctx_idx_v, neg_idx_v,
              crow_a, negrows_a, crow_b, negrows_b,
              negacc_v, sem_a, sem_b):
    wid = lax.axis_index("s") * NC + lax.axis_index("c")
    wbase = wid * BPW

    pltpu.sync_copy(ctx_hbm.at[pl.ds(wbase, BPW)], ctx_idx_v)
    pltpu.sync_copy(neg_hbm.at[pl.ds(wbase * NNEG, BPW * NNEG)], neg_idx_v)
    _transform_ref(ctx_idx_v, BPW)
    _transform_ref(neg_idx_v, BPW * NNEG)

    def copies(ch, crow, negrows, sem):
        cps = [pltpu.make_async_copy(
            wc_hbm.at[ctx_idx_v.at[pl.ds(ch * CHUNK, CHUNK)]], crow, sem)]
        for j in range(NIDX // 128):
            cps.append(pltpu.make_async_copy(
                wc_hbm.at[neg_idx_v.at[pl.ds(ch * NIDX + j * 128, 128)]],
                negrows.at[pl.ds(j * 128, 128)], sem))
        return cps

    def issue(ch, crow, negrows, sem):
        for cp in copies(ch, crow, negrows, sem):
            cp.start()

    def wait(ch, crow, negrows, sem):
        for cp in copies(ch, crow, negrows, sem):
            cp.wait()

    def compute(ch, crow, negrows):
        base = wbase + ch * CHUNK

        def row_body(r, _):
            nbase = r * NNEG
            for k in range(DIM // L):
                acc = negrows[nbase, pl.ds(k * L, L)]
                for n in range(1, NNEG):
                    acc = acc + negrows[nbase + n, pl.ds(k * L, L)]
                negacc_v[r, pl.ds(k * L, L)] = acc
            return 0

        lax.fori_loop(0, CHUNK, row_body, 0)
        pltpu.sync_copy(crow, embc_hbm.at[pl.ds(base, CHUNK)])
        pltpu.sync_copy(negacc_v, negacc_hbm.at[pl.ds(base, CHUNK)])

    issue(0, crow_a, negrows_a, sem_a)
    issue(1, crow_b, negrows_b, sem_b)

    def pair_body(g, _):
        ch_a = 2 * g
        wait(ch_a, crow_a, negrows_a, sem_a)
        compute(ch_a, crow_a, negrows_a)
        issue(ch_a + 2, crow_a, negrows_a, sem_a)
        ch_b = 2 * g + 1
        wait(ch_b, crow_b, negrows_b, sem_b)
        compute(ch_b, crow_b, negrows_b)
        issue(ch_b + 2, crow_b, negrows_b, sem_b)
        return 0

    lax.fori_loop(0, NCHUNK // 2 - 1, pair_body, 0)
    wait(NCHUNK - 2, crow_a, negrows_a, sem_a)
    compute(NCHUNK - 2, crow_a, negrows_a)
    wait(NCHUNK - 1, crow_b, negrows_b, sem_b)
    compute(NCHUNK - 1, crow_b, negrows_b)


def _sc2_body(tgt_hbm, embc_hbm, negacc_hbm, wt_hbm,
              pos_hbm, negsum_hbm,
              tgt_idx_v,
              trow_a, crow_a, nacc_a, trow_b, crow_b, nacc_b,
              pos_out_v, negsum_out_v, fold_v, out16p_v, out16n_v,
              sem_a, sem_b):
    wid = lax.axis_index("s") * NC + lax.axis_index("c")
    wbase = wid * BPW

    pltpu.sync_copy(tgt_hbm.at[pl.ds(wbase, BPW)], tgt_idx_v)
    _transform_ref(tgt_idx_v, BPW)

    def copies(ch, trow, crow, nacc, sem):
        base = wbase + ch * CHUNK
        return [
            pltpu.make_async_copy(
                wt_hbm.at[tgt_idx_v.at[pl.ds(ch * CHUNK, CHUNK)]], trow, sem),
            pltpu.make_async_copy(embc_hbm.at[pl.ds(base, CHUNK)], crow, sem),
            pltpu.make_async_copy(negacc_hbm.at[pl.ds(base, CHUNK)], nacc,
                                  sem),
        ]

    def issue(ch, trow, crow, nacc, sem):
        for cp in copies(ch, trow, crow, nacc, sem):
            cp.start()

    def wait(ch, trow, crow, nacc, sem):
        for cp in copies(ch, trow, crow, nacc, sem):
            cp.wait()

    def compute(ch, trow, crow, nacc):
        for g2 in range(CHUNK // L):
            def row_body(r16, _):
                r = g2 * L + r16
                t = [trow[r, pl.ds(k * L, L)] for k in range(DIM // L)]
                pv = t[0] * crow[r, pl.ds(0, L)]
                nv = t[0] * nacc[r, pl.ds(0, L)]
                for k in range(1, DIM // L):
                    pv = pv + t[k] * crow[r, pl.ds(k * L, L)]
                    nv = nv + t[k] * nacc[r, pl.ds(k * L, L)]
                for d in (8, 4, 2, 1):
                    fold_v[pl.ds(0, L)] = pv
                    pv = pv + fold_v[pl.ds(d, L)]
                for d in (8, 4, 2, 1):
                    fold_v[pl.ds(0, L)] = nv
                    nv = nv + fold_v[pl.ds(d, L)]
                out16p_v[pl.ds(r16, L)] = pv
                out16n_v[pl.ds(r16, L)] = nv
                return 0

            lax.fori_loop(0, L, row_body, 0)
            off = ch * CHUNK + g2 * L
            pos_out_v[pl.ds(off, L)] = out16p_v[pl.ds(0, L)]
            negsum_out_v[pl.ds(off, L)] = out16n_v[pl.ds(0, L)]

    issue(0, trow_a, crow_a, nacc_a, sem_a)
    issue(1, trow_b, crow_b, nacc_b, sem_b)

    def pair_body(g, _):
        ch_a = 2 * g
        wait(ch_a, trow_a, crow_a, nacc_a, sem_a)
        compute(ch_a, trow_a, crow_a, nacc_a)
        issue(ch_a + 2, trow_a, crow_a, nacc_a, sem_a)
        ch_b = 2 * g + 1
        wait(ch_b, trow_b, crow_b, nacc_b, sem_b)
        compute(ch_b, trow_b, crow_b, nacc_b)
        issue(ch_b + 2, trow_b, crow_b, nacc_b, sem_b)
        return 0

    lax.fori_loop(0, NCHUNK // 2 - 1, pair_body, 0)
    wait(NCHUNK - 2, trow_a, crow_a, nacc_a, sem_a)
    compute(NCHUNK - 2, trow_a, crow_a, nacc_a)
    wait(NCHUNK - 1, trow_b, crow_b, nacc_b, sem_b)
    compute(NCHUNK - 1, trow_b, crow_b, nacc_b)

    pltpu.sync_copy(pos_out_v, pos_hbm.at[pl.ds(wbase, BPW)])
    pltpu.sync_copy(negsum_out_v, negsum_hbm.at[pl.ds(wbase, BPW)])


def _mesh():
    return plsc.VectorSubcoreMesh(core_axis_name="c", subcore_axis_name="s",
                                  num_cores=NC, num_subcores=NS)


def _make_sc1():
    return pl.kernel(
        _sc1_body,
        out_type=(
            jax.ShapeDtypeStruct((BATCH, DIM), jnp.float32),
            jax.ShapeDtypeStruct((BATCH, DIM), jnp.float32),
        ),
        mesh=_mesh(),
        compiler_params=pltpu.CompilerParams(use_tc_tiling_on_sc=False),
        scratch_types=[
            pltpu.VMEM((BPW,), jnp.int32),
            pltpu.VMEM((BPW * NNEG,), jnp.int32),
            pltpu.VMEM((CHUNK, DIM), jnp.float32),
            pltpu.VMEM((NIDX, DIM), jnp.float32),
            pltpu.VMEM((CHUNK, DIM), jnp.float32),
            pltpu.VMEM((NIDX, DIM), jnp.float32),
            pltpu.VMEM((CHUNK, DIM), jnp.float32),
            pltpu.SemaphoreType.DMA,
            pltpu.SemaphoreType.DMA,
        ],
    )


def _make_sc2():
    return pl.kernel(
        _sc2_body,
        out_type=(
            jax.ShapeDtypeStruct((BATCH,), jnp.float32),
            jax.ShapeDtypeStruct((BATCH,), jnp.float32),
        ),
        mesh=_mesh(),
        compiler_params=pltpu.CompilerParams(use_tc_tiling_on_sc=False),
        scratch_types=[
            pltpu.VMEM((BPW,), jnp.int32),
            pltpu.VMEM((CHUNK, DIM), jnp.float32),
            pltpu.VMEM((CHUNK, DIM), jnp.float32),
            pltpu.VMEM((CHUNK, DIM), jnp.float32),
            pltpu.VMEM((CHUNK, DIM), jnp.float32),
            pltpu.VMEM((CHUNK, DIM), jnp.float32),
            pltpu.VMEM((CHUNK, DIM), jnp.float32),
            pltpu.VMEM((BPW,), jnp.float32),
            pltpu.VMEM((BPW,), jnp.float32),
            pltpu.VMEM((32,), jnp.float32),
            pltpu.VMEM((32,), jnp.float32),
            pltpu.VMEM((32,), jnp.float32),
            pltpu.SemaphoreType.DMA,
            pltpu.SemaphoreType.DMA,
        ],
    )


def _loss_kernel(p_ref, n_ref, out_ref):
    p = p_ref[...]
    q = -n_ref[...]
    lsp = jnp.minimum(p, 0.0) - jnp.log1p(jnp.exp(-jnp.abs(p)))
    lsq = jnp.minimum(q, 0.0) - jnp.log1p(jnp.exp(-jnp.abs(q)))
    out_ref[...] = jnp.full((1, 1), -(jnp.sum(lsp) + jnp.sum(lsq)),
                            jnp.float32)


@jax.jit
def kernel(target_word, context_word, negative_example, W_target, W_context):
    neg_flat = negative_example.reshape(BATCH * NNEG)
    wc64 = _relayout_table(W_context)
    embc, negacc = _make_sc1()(context_word.astype(jnp.int32),
                               neg_flat.astype(jnp.int32), wc64)
    wt64 = _relayout_table(W_target)
    pos, negsum = _make_sc2()(target_word.astype(jnp.int32),
                              embc, negacc, wt64)
    loss = pl.pallas_call(
        _loss_kernel,
        out_shape=jax.ShapeDtypeStruct((1, 1), jnp.float32),
    )(pos.reshape(128, 128), negsum.reshape(128, 128))
    return loss[0, 0]


# speedup vs baseline: 1.2099x; 1.2099x over previous
import jax
import jax.numpy as jnp
from jax import lax
from jax.experimental import pallas as pl
from jax.experimental.pallas import tpu as pltpu
from jax.experimental.pallas import tpu_sc as plsc

VOCAB = 1000000
DIM = 64
BATCH = 16384
NNEG = 20

LOG2C = 14
C = 1 << LOG2C
CB = 2 * C
NBLK = (VOCAB + CB - 1) // CB
VR = NBLK * C
TROWS = 2 * VR

NC = 2
NS = 16
L = 16
NW = NC * NS
BPW = BATCH // NW
CHUNK = 64
NCHUNK = BPW // CHUNK
NIDX = CHUNK * NNEG


Q = CB // 4


def _tr_kernel(x_ref, pe_ref, po_ref, o_ref):
    x = x_ref[...]
    dn = (((0,), (0,)), ((), ()))
    xcat = jnp.concatenate([x[:, m * Q:(m + 1) * Q] for m in range(4)],
                           axis=0)
    ye = lax.dot_general(xcat, pe_ref[...], dn,
                         preferred_element_type=jnp.float32)
    yo = lax.dot_general(xcat, po_ref[...], dn,
                         preferred_element_type=jnp.float32)
    ue = lax.bitcast_convert_type(ye.astype(jnp.bfloat16),
                                  jnp.uint16).astype(jnp.uint32)
    uo = lax.bitcast_convert_type(yo.astype(jnp.bfloat16),
                                  jnp.uint16).astype(jnp.uint32)
    o_ref[...] = lax.bitcast_convert_type(
        (uo << 16) | ue, jnp.int32)


def _relayout_table(w):
    rows = jnp.arange(4 * DIM)[:, None]
    q = jnp.arange(128)[None, :]
    src_e = (q // 32) * 64 + 2 * (q % 32)
    pe = (rows == src_e).astype(jnp.float32)
    po = (rows == src_e + 1).astype(jnp.float32)
    w128 = pl.pallas_call(
        _tr_kernel,
        grid=(NBLK,),
        in_specs=[pl.BlockSpec((DIM, CB), lambda i: (0, i)),
                  pl.BlockSpec((4 * DIM, 128), lambda i: (0, 0)),
                  pl.BlockSpec((4 * DIM, 128), lambda i: (0, 0))],
        out_specs=pl.BlockSpec((Q, 128), lambda i: (i, 0)),
        out_shape=jax.ShapeDtypeStruct((NBLK * Q, 128), jnp.int32),
        compiler_params=pltpu.CompilerParams(
            vmem_limit_bytes=100 * 1024 * 1024),
    )(w.T, pe, po)
    return w128.reshape(TROWS, DIM // 2)


def _transform_ref(ref, n):
    def body(j, _):
        v = ref[pl.ds(j * L, L)]
        g = (((v >> (LOG2C + 1)) << (LOG2C + 1))
             | ((v & (C // 2 - 1)) << 2) | ((v >> (LOG2C - 1)) & 3))
        ref[pl.ds(j * L, L)] = g
        return 0
    lax.fori_loop(0, n // L, body, 0)


def _sc_kernel_body(tgt_hbm, ctx_hbm, neg_hbm, wt_hbm, wc_hbm,
                    pos_hbm, negsum_hbm,
                    tgt_idx_v, ctx_idx_v, neg_idx_v,
                    trow_a, crow_a, negrows_a,
                    trow_b, crow_b, negrows_b,
                    pos_out_v, negsum_out_v, fold_v, out16p_v, out16n_v,
                    sem_a, sem_b):
    wid = lax.axis_index("s") * NC + lax.axis_index("c")
    wbase = wid * BPW

    pltpu.sync_copy(tgt_hbm.at[pl.ds(wbase, BPW)], tgt_idx_v)
    pltpu.sync_copy(ctx_hbm.at[pl.ds(wbase, BPW)], ctx_idx_v)
    pltpu.sync_copy(neg_hbm.at[pl.ds(wbase * NNEG, BPW * NNEG)], neg_idx_v)
    _transform_ref(tgt_idx_v, BPW)
    _transform_ref(ctx_idx_v, BPW)
    _transform_ref(neg_idx_v, BPW * NNEG)

    def copies(ch, trow, crow, negrows, sem):
        cps = [
            pltpu.make_async_copy(
                wt_hbm.at[tgt_idx_v.at[pl.ds(ch * CHUNK, CHUNK)]], trow, sem),
            pltpu.make_async_copy(
                wc_hbm.at[ctx_idx_v.at[pl.ds(ch * CHUNK, CHUNK)]], crow, sem),
        ]
        for j in range(NIDX // 128):
            cps.append(pltpu.make_async_copy(
                wc_hbm.at[neg_idx_v.at[pl.ds(ch * NIDX + j * 128, 128)]],
                negrows.at[pl.ds(j * 128, 128)], sem))
        return cps

    def issue(ch, trow, crow, negrows, sem):
        for cp in copies(ch, trow, crow, negrows, sem):
            cp.start()

    def wait(ch, trow, crow, negrows, sem):
        for cp in copies(ch, trow, crow, negrows, sem):
            cp.wait()

    def compute(ch, trow, crow, negrows):
        for g2 in range(CHUNK // L):
            def unpack2(w):
                even = lax.bitcast_convert_type(w << 16, jnp.float32)
                odd = lax.bitcast_convert_type(
                    w & jnp.int32(-65536), jnp.float32)
                return even, odd

            def row_body(r16, _):
                r = g2 * L + r16
                t = []
                c = []
                for k in range(DIM // (2 * L)):
                    t.extend(unpack2(trow[r, pl.ds(k * L, L)]))
                    c.extend(unpack2(crow[r, pl.ds(k * L, L)]))
                pv = t[0] * c[0]
                for k in range(1, len(t)):
                    pv = pv + t[k] * c[k]
                nbase = r * NNEG
                acc = list(unpack2(negrows[nbase, pl.ds(0, L)]))
                acc.extend(unpack2(negrows[nbase, pl.ds(L, L)]))
                for n in range(1, NNEG):
                    for k in range(DIM // (2 * L)):
                        a, b = unpack2(negrows[nbase + n, pl.ds(k * L, L)])
                        acc[2 * k] = acc[2 * k] + a
                        acc[2 * k + 1] = acc[2 * k + 1] + b
                nv = acc[0] * t[0]
                for k in range(1, len(t)):
                    nv = nv + acc[k] * t[k]
                for d in (8, 4, 2, 1):
                    fold_v[pl.ds(0, L)] = pv
                    pv = pv + fold_v[pl.ds(d, L)]
                for d in (8, 4, 2, 1):
                    fold_v[pl.ds(0, L)] = nv
                    nv = nv + fold_v[pl.ds(d, L)]
                out16p_v[pl.ds(r16, L)] = pv
                out16n_v[pl.ds(r16, L)] = nv
                return 0

            lax.fori_loop(0, L, row_body, 0)
            off = ch * CHUNK + g2 * L
            pos_out_v[pl.ds(off, L)] = out16p_v[pl.ds(0, L)]
            negsum_out_v[pl.ds(off, L)] = out16n_v[pl.ds(0, L)]

    issue(0, trow_a, crow_a, negrows_a, sem_a)
    issue(1, trow_b, crow_b, negrows_b, sem_b)

    def pair_body(g, _):
        ch_a = 2 * g
        wait(ch_a, trow_a, crow_a, negrows_a, sem_a)
        compute(ch_a, trow_a, crow_a, negrows_a)
        issue(ch_a + 2, trow_a, crow_a, negrows_a, sem_a)
        ch_b = 2 * g + 1
        wait(ch_b, trow_b, crow_b, negrows_b, sem_b)
        compute(ch_b, trow_b, crow_b, negrows_b)
        issue(ch_b + 2, trow_b, crow_b, negrows_b, sem_b)
        return 0

    lax.fori_loop(0, NCHUNK // 2 - 1, pair_body, 0)
    last_a = NCHUNK - 2
    wait(last_a, trow_a, crow_a, negrows_a, sem_a)
    compute(last_a, trow_a, crow_a, negrows_a)
    last_b = NCHUNK - 1
    wait(last_b, trow_b, crow_b, negrows_b, sem_b)
    compute(last_b, trow_b, crow_b, negrows_b)

    pltpu.sync_copy(pos_out_v, pos_hbm.at[pl.ds(wbase, BPW)])
    pltpu.sync_copy(negsum_out_v, negsum_hbm.at[pl.ds(wbase, BPW)])


def _make_sc_call():
    mesh = plsc.VectorSubcoreMesh(core_axis_name="c", subcore_axis_name="s",
                                  num_cores=NC, num_subcores=NS)
    return pl.kernel(
        _sc_kernel_body,
        out_type=(
            jax.ShapeDtypeStruct((BATCH,), jnp.float32),
            jax.ShapeDtypeStruct((BATCH,), jnp.float32),
        ),
        mesh=mesh,
        compiler_params=pltpu.CompilerParams(use_tc_tiling_on_sc=False),
        scratch_types=[
            pltpu.VMEM((BPW,), jnp.int32),
            pltpu.VMEM((BPW,), jnp.int32),
            pltpu.VMEM((BPW * NNEG,), jnp.int32),
            pltpu.VMEM((CHUNK, DIM // 2), jnp.int32),
            pltpu.VMEM((CHUNK, DIM // 2), jnp.int32),
            pltpu.VMEM((NIDX, DIM // 2), jnp.int32),
            pltpu.VMEM((CHUNK, DIM // 2), jnp.int32),
            pltpu.VMEM((CHUNK, DIM // 2), jnp.int32),
            pltpu.VMEM((NIDX, DIM // 2), jnp.int32),
            pltpu.VMEM((BPW,), jnp.float32),
            pltpu.VMEM((BPW,), jnp.float32),
            pltpu.VMEM((32,), jnp.float32),
            pltpu.VMEM((32,), jnp.float32),
            pltpu.VMEM((32,), jnp.float32),
            pltpu.SemaphoreType.DMA,
            pltpu.SemaphoreType.DMA,
        ],
    )


def _loss_kernel(p_ref, n_ref, out_ref):
    p = p_ref[...]
    q = -n_ref[...]
    lsp = jnp.minimum(p, 0.0) - jnp.log1p(jnp.exp(-jnp.abs(p)))
    lsq = jnp.minimum(q, 0.0) - jnp.log1p(jnp.exp(-jnp.abs(q)))
    out_ref[...] = jnp.full((1, 1), -(jnp.sum(lsp) + jnp.sum(lsq)),
                            jnp.float32)


@jax.jit
def kernel(target_word, context_word, negative_example, W_target, W_context):
    neg_flat = negative_example.reshape(BATCH * NNEG)
    wt64 = _relayout_table(W_target)
    wc64 = _relayout_table(W_context)
    sc = _make_sc_call()
    pos, negsum = sc(target_word.astype(jnp.int32),
                     context_word.astype(jnp.int32),
                     neg_flat.astype(jnp.int32),
                     wt64, wc64)
    loss = pl.pallas_call(
        _loss_kernel,
        out_shape=jax.ShapeDtypeStruct((1, 1), jnp.float32),
    )(pos.reshape(128, 128), negsum.reshape(128, 128))
    return loss[0, 0]
